# baseline (speedup 1.0000x reference)
import functools

import jax
import jax.numpy as jnp
from jax import lax
from jax.experimental import pallas as pl
from jax.experimental.pallas import tpu as pltpu
from jax.experimental.pallas import tpu_sc as plsc

N_NODES = 10000
N_EDGES = 320000
D_FEAT = 128
EMBED_DIM = 128
BATCH = 4096
NUM_FIELDS = 26

NC = 2
NS = 16
NW = NC * NS
L = 16

CH = 80
EPW = CH * 128
EPAD = NW * EPW
NBUF = 2
CHH = CH // 2
NP = 10240
RPT = NP // NS
DUMMY = N_NODES + 16
XROWS = BATCH * NUM_FIELDS // 128
XPW = XROWS // NW

_MESH = plsc.VectorSubcoreMesh(core_axis_name="c", subcore_axis_name="s")


@functools.partial(
    pl.kernel,
    out_type=jax.ShapeDtypeStruct((NW * NP,), jnp.float32),
    mesh=_MESH,
    scratch_types=[
        pltpu.VMEM((2, CHH, 128), jnp.int32),
        pltpu.VMEM((NP,), jnp.float32),
    ],
    compiler_params=pltpu.CompilerParams(needs_layout_passes=False),
)
def _sc_degree(dst_hbm, out_hbm, idx_v, deg_v):
    c = lax.axis_index("c")
    s = lax.axis_index("s")
    wid = c * NS + s
    pltpu.sync_copy(dst_hbm.at[wid], idx_v)

    zeros = jnp.zeros((L,), jnp.float32)

    def _zero(i, carry):
        deg_v[pl.ds(i * L, L)] = zeros
        return carry

    lax.fori_loop(0, NP // L, _zero, 0)

    ones = jnp.ones((L,), jnp.float32)

    for h in range(2):
        def _chunk(j, carry):
            def _sub(k, carry2):
                idx16 = idx_v[h, j, pl.ds(k * L, L)]
                plsc.addupdate_scatter(deg_v, [idx16], ones)
                return carry2

            return lax.fori_loop(0, 128 // L, _sub, carry)

        lax.fori_loop(0, CHH, _chunk, 0)
    pltpu.sync_copy(deg_v, out_hbm.at[pl.ds(wid * NP, NP)])


def _dinv_col(degp_blk):
    ones = jnp.ones((NW, 1), jnp.float32)
    deg = lax.dot_general(degp_blk, ones, (((0,), (0,)), ((), ())),
                          preferred_element_type=jnp.float32) + 1.0
    return lax.rsqrt(deg)


def _tc_prep_body(feat_ref, w_ref, degp_ref, y_ref):
    xw = jnp.dot(feat_ref[...], w_ref[...], preferred_element_type=jnp.float32)
    y_ref[...] = xw * _dinv_col(degp_ref[...])


_TCR = 1024

_tc_prep = pl.pallas_call(
    _tc_prep_body,
    grid=(NP // _TCR,),
    in_specs=[
        pl.BlockSpec((_TCR, D_FEAT), lambda i: (i, 0)),
        pl.BlockSpec((D_FEAT, EMBED_DIM), lambda i: (0, 0)),
        pl.BlockSpec((NW, _TCR), lambda i: (0, i)),
    ],
    out_specs=pl.BlockSpec((_TCR, EMBED_DIM), lambda i: (i, 0)),
    out_shape=jax.ShapeDtypeStruct((NP, EMBED_DIM), jnp.float32),
)


@functools.partial(
    pl.kernel,
    out_type=jax.ShapeDtypeStruct((NC, NP, EMBED_DIM), jnp.float32),
    mesh=_MESH,
    scratch_types=[
        pltpu.VMEM((CHH, 128), jnp.int32),
        pltpu.VMEM((CHH, 128), jnp.int32),
        pltpu.VMEM((128, EMBED_DIM), jnp.float32),
        pltpu.VMEM_SHARED((NP, EMBED_DIM), jnp.float32),
    ],
)
def _sc_scatter(src_hbm, dst_hbm, y_hbm, zeros_hbm, out_hbm,
                src_v, dst_v, buf, acc_sh):
    c = lax.axis_index("c")
    s = lax.axis_index("s")
    wid = c * NS + s
    pltpu.sync_copy(zeros_hbm, acc_sh.at[pl.ds(s * RPT, RPT)])
    plsc.subcore_barrier()

    for h in range(2):
        pltpu.sync_copy(src_hbm.at[wid, h], src_v)
        pltpu.sync_copy(dst_hbm.at[wid, h], dst_v)

        def _chunk(j, carry):
            pltpu.sync_copy(y_hbm.at[src_v.at[j]], buf)
            pltpu.sync_copy(buf, acc_sh.at[dst_v.at[j]], add=True)
            return carry

        lax.fori_loop(0, CHH, _chunk, 0)

    plsc.subcore_barrier()
    pltpu.sync_copy(acc_sh.at[pl.ds(s * RPT, RPT)],
                    out_hbm.at[c, pl.ds(s * RPT, RPT)])


def _tc_combine_body(acc_ref, y_ref, degp_ref, b_ref, out_ref):
    acc = acc_ref[0] + acc_ref[1]
    dinv = _dinv_col(degp_ref[...])
    out_ref[...] = dinv * (acc + y_ref[...]) + b_ref[...][None, :]


_tc_combine = pl.pallas_call(
    _tc_combine_body,
    grid=(NP // _TCR,),
    in_specs=[
        pl.BlockSpec((NC, _TCR, EMBED_DIM), lambda i: (0, i, 0)),
        pl.BlockSpec((_TCR, EMBED_DIM), lambda i: (i, 0)),
        pl.BlockSpec((NW, _TCR), lambda i: (0, i)),
        pl.BlockSpec((EMBED_DIM,), lambda i: (0,)),
    ],
    out_specs=pl.BlockSpec((_TCR, EMBED_DIM), lambda i: (i, 0)),
    out_shape=jax.ShapeDtypeStruct((NP, EMBED_DIM), jnp.float32),
)


@functools.partial(
    pl.kernel,
    out_type=jax.ShapeDtypeStruct((XROWS, 128, EMBED_DIM), jnp.float32),
    mesh=_MESH,
    scratch_types=[
        pltpu.VMEM((XPW, 128), jnp.int32),
        pltpu.VMEM((128, EMBED_DIM), jnp.float32),
    ],
)
def _sc_gather(emb_hbm, x_hbm, out_hbm, idx_v, rows_v):
    c = lax.axis_index("c")
    s = lax.axis_index("s")
    wid = c * NS + s
    pltpu.sync_copy(x_hbm.at[wid], idx_v)

    def _chunk(j, carry):
        pltpu.sync_copy(emb_hbm.at[idx_v.at[j]], rows_v)
        pltpu.sync_copy(rows_v, out_hbm.at[wid * XPW + j])
        return carry

    lax.fori_loop(0, XPW, _chunk, 0)


def kernel(x, features, edge_index, W, b):
    src = edge_index[0].astype(jnp.int32)
    dst = edge_index[1].astype(jnp.int32)
    npad = EPAD - N_EDGES
    src2d = jnp.concatenate(
        [src, jnp.zeros((npad,), jnp.int32)]).reshape(NW, 2, CHH, 128)
    dst2d = jnp.concatenate(
        [dst, jnp.full((npad,), DUMMY, jnp.int32)]).reshape(NW, 2, CHH, 128)
    x2d = x.astype(jnp.int32).reshape(NW, XPW, 128)
    zeros_blk = jnp.zeros((RPT, EMBED_DIM), jnp.float32)
    feat_pad = jnp.concatenate(
        [features, jnp.zeros((NP - N_NODES, D_FEAT), jnp.float32)])

    degp = _sc_degree(dst2d).reshape(NW, NP)
    y = _tc_prep(feat_pad, W, degp)
    acc = _sc_scatter(src2d, dst2d, y, zeros_blk)
    emb = _tc_combine(acc, y, degp, b)
    out = _sc_gather(emb, x2d)
    return out.reshape(BATCH, NUM_FIELDS, EMBED_DIM)


# speedup vs baseline: 1.0082x; 1.0082x over previous
import functools

import jax
import jax.numpy as jnp
from jax import lax
from jax.experimental import pallas as pl
from jax.experimental.pallas import tpu as pltpu
from jax.experimental.pallas import tpu_sc as plsc

N_NODES = 10000
N_EDGES = 320000
D_FEAT = 128
EMBED_DIM = 128
BATCH = 4096
NUM_FIELDS = 26

NC = 2
NS = 16
NW = NC * NS
L = 16

CH = 80
EPW = CH * 128
EPAD = NW * EPW
NBUF = 2
CHH = CH // 2
NP = 10240
RPT = NP // NS
DUMMY = N_NODES + 16
XROWS = BATCH * NUM_FIELDS // 128
XPW = XROWS // NW

_MESH = plsc.VectorSubcoreMesh(core_axis_name="c", subcore_axis_name="s")


@functools.partial(
    pl.kernel,
    out_type=jax.ShapeDtypeStruct((NW * NP,), jnp.float32),
    mesh=_MESH,
    scratch_types=[
        pltpu.VMEM((2, CHH, 128), jnp.int32),
        pltpu.VMEM((NP,), jnp.float32),
    ],
    compiler_params=pltpu.CompilerParams(needs_layout_passes=False),
)
def _sc_degree(dst_hbm, out_hbm, idx_v, deg_v):
    c = lax.axis_index("c")
    s = lax.axis_index("s")
    wid = c * NS + s
    pltpu.sync_copy(dst_hbm.at[wid], idx_v)

    zeros = jnp.zeros((L,), jnp.float32)

    def _zero(i, carry):
        deg_v[pl.ds(i * L, L)] = zeros
        return carry

    lax.fori_loop(0, NP // L, _zero, 0)

    ones = jnp.ones((L,), jnp.float32)

    for h in range(2):
        def _chunk(j, carry):
            def _sub(k, carry2):
                idx16 = idx_v[h, j, pl.ds(k * L, L)]
                plsc.addupdate_scatter(deg_v, [idx16], ones)
                return carry2

            return lax.fori_loop(0, 128 // L, _sub, carry)

        lax.fori_loop(0, CHH, _chunk, 0)
    pltpu.sync_copy(deg_v, out_hbm.at[pl.ds(wid * NP, NP)])


def _dinv_col(degp_blk):
    ones = jnp.ones((NW, 1), jnp.float32)
    deg = lax.dot_general(degp_blk, ones, (((0,), (0,)), ((), ())),
                          preferred_element_type=jnp.float32) + 1.0
    return lax.rsqrt(deg)


def _tc_prep_body(feat_ref, w_ref, degp_ref, y_ref):
    xw = jnp.dot(feat_ref[...], w_ref[...], preferred_element_type=jnp.float32)
    y_ref[...] = xw * _dinv_col(degp_ref[...])


_TCR = 1024

_tc_prep = pl.pallas_call(
    _tc_prep_body,
    grid=(NP // _TCR,),
    in_specs=[
        pl.BlockSpec((_TCR, D_FEAT), lambda i: (i, 0)),
        pl.BlockSpec((D_FEAT, EMBED_DIM), lambda i: (0, 0)),
        pl.BlockSpec((NW, _TCR), lambda i: (0, i)),
    ],
    out_specs=pl.BlockSpec((_TCR, EMBED_DIM), lambda i: (i, 0)),
    out_shape=jax.ShapeDtypeStruct((NP, EMBED_DIM), jnp.float32),
)


@functools.partial(
    pl.kernel,
    out_type=jax.ShapeDtypeStruct((NC, NP, EMBED_DIM), jnp.float32),
    mesh=_MESH,
    scratch_types=[
        pltpu.VMEM((CHH, 128), jnp.int32),
        pltpu.VMEM((CHH, 128), jnp.int32),
        pltpu.VMEM((128, EMBED_DIM), jnp.float32),
        pltpu.VMEM_SHARED((NP, EMBED_DIM), jnp.float32),
    ],
)
def _sc_scatter(src_hbm, dst_hbm, y_hbm, zeros_hbm, out_hbm,
                src_v, dst_v, buf, acc_sh):
    c = lax.axis_index("c")
    s = lax.axis_index("s")
    wid = c * NS + s
    pltpu.sync_copy(zeros_hbm, acc_sh.at[pl.ds(s * RPT, RPT)])
    plsc.subcore_barrier()

    for h in range(2):
        pltpu.sync_copy(src_hbm.at[wid, h], src_v)
        pltpu.sync_copy(dst_hbm.at[wid, h], dst_v)

        def _chunk(j, carry):
            pltpu.sync_copy(y_hbm.at[src_v.at[j]], buf)
            pltpu.sync_copy(buf, acc_sh.at[dst_v.at[j]], add=True)
            return carry

        lax.fori_loop(0, CHH, _chunk, 0)

    plsc.subcore_barrier()
    pltpu.sync_copy(acc_sh.at[pl.ds(s * RPT, RPT)],
                    out_hbm.at[c, pl.ds(s * RPT, RPT)])


def _tc_combine_body(acc_ref, y_ref, degp_ref, b_ref, out_ref):
    acc = acc_ref[0] + acc_ref[1]
    dinv = _dinv_col(degp_ref[...])
    out_ref[...] = dinv * (acc + y_ref[...]) + b_ref[...][None, :]


_tc_combine = pl.pallas_call(
    _tc_combine_body,
    grid=(NP // _TCR,),
    in_specs=[
        pl.BlockSpec((NC, _TCR, EMBED_DIM), lambda i: (0, i, 0)),
        pl.BlockSpec((_TCR, EMBED_DIM), lambda i: (i, 0)),
        pl.BlockSpec((NW, _TCR), lambda i: (0, i)),
        pl.BlockSpec((EMBED_DIM,), lambda i: (0,)),
    ],
    out_specs=pl.BlockSpec((_TCR, EMBED_DIM), lambda i: (i, 0)),
    out_shape=jax.ShapeDtypeStruct((NP, EMBED_DIM), jnp.float32),
)


@functools.partial(
    pl.kernel,
    out_type=jax.ShapeDtypeStruct((XROWS, 128, EMBED_DIM), jnp.float32),
    mesh=_MESH,
    scratch_types=[
        pltpu.VMEM((XPW, 128), jnp.int32),
        pltpu.VMEM((128, EMBED_DIM), jnp.float32),
    ],
)
def _sc_gather(emb_hbm, x_hbm, out_hbm, idx_v, rows_v):
    c = lax.axis_index("c")
    s = lax.axis_index("s")
    wid = c * NS + s
    pltpu.sync_copy(x_hbm.at[wid], idx_v)

    def _chunk(j, carry):
        pltpu.sync_copy(emb_hbm.at[idx_v.at[j]], rows_v)
        pltpu.sync_copy(rows_v, out_hbm.at[wid * XPW + j])
        return carry

    lax.fori_loop(0, XPW, _chunk, 0)


def kernel(x, features, edge_index, W, b):
    src = edge_index[0].astype(jnp.int32)
    dst = edge_index[1].astype(jnp.int32)
    npad = EPAD - N_EDGES
    pad_dst = N_NODES + (lax.iota(jnp.int32, npad) % (NP - N_NODES))
    src2d = jnp.concatenate(
        [src, jnp.zeros((npad,), jnp.int32)]).reshape(NW, 2, CHH, 128)
    dst2d = jnp.concatenate(
        [dst, pad_dst]).reshape(NW, 2, CHH, 128)
    x2d = x.astype(jnp.int32).reshape(NW, XPW, 128)
    zeros_blk = jnp.zeros((RPT, EMBED_DIM), jnp.float32)
    feat_pad = jnp.concatenate(
        [features, jnp.zeros((NP - N_NODES, D_FEAT), jnp.float32)])

    degp = _sc_degree(dst2d).reshape(NW, NP)
    y = _tc_prep(feat_pad, W, degp)
    acc = _sc_scatter(src2d, dst2d, y, zeros_blk)
    emb = _tc_combine(acc, y, degp, b)
    out = _sc_gather(emb, x2d)
    return out.reshape(BATCH, NUM_FIELDS, EMBED_DIM)


# speedup vs baseline: 1.7236x; 1.7095x over previous
import functools

import jax
import jax.numpy as jnp
from jax import lax
from jax.experimental import pallas as pl
from jax.experimental.pallas import tpu as pltpu
from jax.experimental.pallas import tpu_sc as plsc

N_NODES = 10000
N_EDGES = 320000
D_FEAT = 128
EMBED_DIM = 128
BATCH = 4096
NUM_FIELDS = 26

NC = 2
NS = 16
NW = NC * NS
L = 16

CH = 80
EPW = CH * 128
EPAD = NW * EPW
NBUF = 2
CHH = CH // 2
NP = 10240
RPT = NP // NS
DUMMY = N_NODES + 16
XROWS = BATCH * NUM_FIELDS // 128
XPW = XROWS // NW

_MESH = plsc.VectorSubcoreMesh(core_axis_name="c", subcore_axis_name="s")


@functools.partial(
    pl.kernel,
    out_type=jax.ShapeDtypeStruct((NW * NP,), jnp.float32),
    mesh=_MESH,
    scratch_types=[
        pltpu.VMEM((2, CHH, 128), jnp.int32),
        pltpu.VMEM((NP,), jnp.float32),
    ],
    compiler_params=pltpu.CompilerParams(needs_layout_passes=False),
)
def _sc_degree(dst_hbm, out_hbm, idx_v, deg_v):
    c = lax.axis_index("c")
    s = lax.axis_index("s")
    wid = c * NS + s
    pltpu.sync_copy(dst_hbm.at[wid], idx_v)

    zeros = jnp.zeros((L,), jnp.float32)

    def _zero(i, carry):
        deg_v[pl.ds(i * L, L)] = zeros
        return carry

    lax.fori_loop(0, NP // L, _zero, 0)

    ones = jnp.ones((L,), jnp.float32)

    for h in range(2):
        def _chunk(j, carry):
            def _sub(k, carry2):
                idx16 = idx_v[h, j, pl.ds(k * L, L)]
                plsc.addupdate_scatter(deg_v, [idx16], ones)
                return carry2

            return lax.fori_loop(0, 128 // L, _sub, carry)

        lax.fori_loop(0, CHH, _chunk, 0)
    pltpu.sync_copy(deg_v, out_hbm.at[pl.ds(wid * NP, NP)])


def _dinv_col(degp_blk):
    ones = jnp.ones((NW, 1), jnp.float32)
    deg = lax.dot_general(degp_blk, ones, (((0,), (0,)), ((), ())),
                          preferred_element_type=jnp.float32) + 1.0
    return lax.rsqrt(deg)


def _tc_prep_body(feat_ref, w_ref, degp_ref, y_ref):
    xw = jnp.dot(feat_ref[...], w_ref[...], preferred_element_type=jnp.float32)
    y_ref[...] = xw * _dinv_col(degp_ref[...])


_TCR = 1024

_tc_prep = pl.pallas_call(
    _tc_prep_body,
    grid=(NP // _TCR,),
    in_specs=[
        pl.BlockSpec((_TCR, D_FEAT), lambda i: (i, 0)),
        pl.BlockSpec((D_FEAT, EMBED_DIM), lambda i: (0, 0)),
        pl.BlockSpec((NW, _TCR), lambda i: (0, i)),
    ],
    out_specs=pl.BlockSpec((_TCR, EMBED_DIM), lambda i: (i, 0)),
    out_shape=jax.ShapeDtypeStruct((NP, EMBED_DIM), jnp.float32),
)


@functools.partial(
    pl.kernel,
    out_type=jax.ShapeDtypeStruct((NC, NP, EMBED_DIM), jnp.float32),
    mesh=_MESH,
    scratch_types=[
        pltpu.VMEM((CHH, 128), jnp.int32),
        pltpu.VMEM((CHH, 128), jnp.int32),
        pltpu.VMEM((128, EMBED_DIM), jnp.float32),
        pltpu.VMEM_SHARED((NP, EMBED_DIM), jnp.float32),
    ],
)
def _sc_scatter(src_hbm, dst_hbm, y_hbm, zeros_hbm, out_hbm,
                src_v, dst_v, buf, acc_sh):
    c = lax.axis_index("c")
    s = lax.axis_index("s")
    wid = c * NS + s
    pltpu.sync_copy(zeros_hbm, acc_sh.at[pl.ds(s * RPT, RPT)])
    plsc.subcore_barrier()

    for h in range(2):
        pltpu.sync_copy(src_hbm.at[wid, h], src_v)
        pltpu.sync_copy(dst_hbm.at[wid, h], dst_v)

        def _chunk(j, carry):
            pltpu.sync_copy(y_hbm.at[src_v.at[j]], buf)
            pltpu.sync_copy(buf, acc_sh.at[dst_v.at[j]], add=True)
            return carry

        lax.fori_loop(0, CHH, _chunk, 0)

    plsc.subcore_barrier()
    pltpu.sync_copy(acc_sh.at[pl.ds(s * RPT, RPT)],
                    out_hbm.at[c, pl.ds(s * RPT, RPT)])


def _tc_combine_body(acc_ref, y_ref, degp_ref, b_ref, out_ref):
    acc = acc_ref[0] + acc_ref[1]
    dinv = _dinv_col(degp_ref[...])
    out_ref[...] = dinv * (acc + y_ref[...]) + b_ref[...][None, :]


_tc_combine = pl.pallas_call(
    _tc_combine_body,
    grid=(NP // _TCR,),
    in_specs=[
        pl.BlockSpec((NC, _TCR, EMBED_DIM), lambda i: (0, i, 0)),
        pl.BlockSpec((_TCR, EMBED_DIM), lambda i: (i, 0)),
        pl.BlockSpec((NW, _TCR), lambda i: (0, i)),
        pl.BlockSpec((EMBED_DIM,), lambda i: (0,)),
    ],
    out_specs=pl.BlockSpec((_TCR, EMBED_DIM), lambda i: (i, 0)),
    out_shape=jax.ShapeDtypeStruct((NP, EMBED_DIM), jnp.float32),
)


@functools.partial(
    pl.kernel,
    out_type=jax.ShapeDtypeStruct((XROWS, 128, EMBED_DIM), jnp.float32),
    mesh=_MESH,
    scratch_types=[
        pltpu.VMEM((XPW, 128), jnp.int32),
        pltpu.VMEM((128, EMBED_DIM), jnp.float32),
    ],
)
def _sc_gather(emb_hbm, x_hbm, out_hbm, idx_v, rows_v):
    c = lax.axis_index("c")
    s = lax.axis_index("s")
    wid = c * NS + s
    pltpu.sync_copy(x_hbm.at[wid], idx_v)

    def _chunk(j, carry):
        pltpu.sync_copy(emb_hbm.at[idx_v.at[j]], rows_v)
        pltpu.sync_copy(rows_v, out_hbm.at[wid * XPW + j])
        return carry

    lax.fori_loop(0, XPW, _chunk, 0)


def kernel(x, features, edge_index, W, b):
    src = edge_index[0].astype(jnp.int32)
    dst = edge_index[1].astype(jnp.int32)
    npad = EPAD - N_EDGES
    pad_dst = N_NODES + (lax.iota(jnp.int32, npad) % (NP - N_NODES))
    pad_src = lax.rem(N_NODES + lax.iota(jnp.int32, npad), NP)
    src2d = jnp.concatenate(
        [src, pad_src]).reshape(NW, 2, CHH, 128)
    dst2d = jnp.concatenate(
        [dst, pad_dst]).reshape(NW, 2, CHH, 128)
    x2d = x.astype(jnp.int32).reshape(NW, XPW, 128)
    zeros_blk = jnp.zeros((RPT, EMBED_DIM), jnp.float32)
    feat_pad = jnp.concatenate(
        [features, jnp.zeros((NP - N_NODES, D_FEAT), jnp.float32)])

    degp = _sc_degree(dst2d).reshape(NW, NP)
    y = _tc_prep(feat_pad, W, degp)
    acc = _sc_scatter(src2d, dst2d, y, zeros_blk)
    emb = _tc_combine(acc, y, degp, b)
    out = _sc_gather(emb, x2d)
    return out.reshape(BATCH, NUM_FIELDS, EMBED_DIM)


# speedup vs baseline: 1.9387x; 1.1248x over previous
import functools

import jax
import jax.numpy as jnp
from jax import lax
from jax.experimental import pallas as pl
from jax.experimental.pallas import tpu as pltpu
from jax.experimental.pallas import tpu_sc as plsc

N_NODES = 10000
N_EDGES = 320000
D_FEAT = 128
EMBED_DIM = 128
BATCH = 4096
NUM_FIELDS = 26

NC = 2
NS = 16
NW = NC * NS
L = 16

CH = 80
EPW = CH * 128
EPAD = NW * EPW
NBUF = 2
CHH = CH // 2
NP = 10240
RPT = NP // NS
DUMMY = N_NODES + 16
XROWS = BATCH * NUM_FIELDS // 128
XPW = XROWS // NW

_MESH = plsc.VectorSubcoreMesh(core_axis_name="c", subcore_axis_name="s")


@functools.partial(
    pl.kernel,
    out_type=jax.ShapeDtypeStruct((NW * NP,), jnp.float32),
    mesh=_MESH,
    scratch_types=[
        pltpu.VMEM((2, CHH, 128), jnp.int32),
        pltpu.VMEM((NP,), jnp.float32),
    ],
    compiler_params=pltpu.CompilerParams(needs_layout_passes=False),
)
def _sc_degree(dst_hbm, out_hbm, idx_v, deg_v):
    c = lax.axis_index("c")
    s = lax.axis_index("s")
    wid = c * NS + s
    pltpu.sync_copy(dst_hbm.at[wid], idx_v)

    zeros = jnp.zeros((L,), jnp.float32)

    def _zero(i, carry):
        deg_v[pl.ds(i * L, L)] = zeros
        return carry

    lax.fori_loop(0, NP // L, _zero, 0)

    ones = jnp.ones((L,), jnp.float32)

    for h in range(2):
        def _chunk(j, carry):
            def _sub(k, carry2):
                idx16 = idx_v[h, j, pl.ds(k * L, L)]
                plsc.addupdate_scatter(deg_v, [idx16], ones)
                return carry2

            return lax.fori_loop(0, 128 // L, _sub, carry)

        lax.fori_loop(0, CHH, _chunk, 0)
    pltpu.sync_copy(deg_v, out_hbm.at[pl.ds(wid * NP, NP)])


def _dinv_col(degp_blk):
    ones = jnp.ones((NW, 1), jnp.float32)
    deg = lax.dot_general(degp_blk, ones, (((0,), (0,)), ((), ())),
                          preferred_element_type=jnp.float32) + 1.0
    return lax.rsqrt(deg)


def _tc_prep_body(feat_ref, w_ref, degp_ref, y_ref):
    xw = jnp.dot(feat_ref[...], w_ref[...], preferred_element_type=jnp.float32)
    y_ref[...] = xw * _dinv_col(degp_ref[...])


_TCR = 1024

_tc_prep = pl.pallas_call(
    _tc_prep_body,
    grid=(NP // _TCR,),
    in_specs=[
        pl.BlockSpec((_TCR, D_FEAT), lambda i: (i, 0)),
        pl.BlockSpec((D_FEAT, EMBED_DIM), lambda i: (0, 0)),
        pl.BlockSpec((NW, _TCR), lambda i: (0, i)),
    ],
    out_specs=pl.BlockSpec((_TCR, EMBED_DIM), lambda i: (i, 0)),
    out_shape=jax.ShapeDtypeStruct((NP, EMBED_DIM), jnp.float32),
)


@functools.partial(
    pl.kernel,
    out_type=jax.ShapeDtypeStruct((NC, NP, EMBED_DIM), jnp.float32),
    mesh=_MESH,
    scratch_types=[
        pltpu.VMEM((CHH, 128), jnp.int32),
        pltpu.VMEM((CHH, 128), jnp.int32),
        pltpu.VMEM((128, EMBED_DIM), jnp.float32),
        pltpu.VMEM((128, EMBED_DIM), jnp.float32),
        pltpu.VMEM_SHARED((NP, EMBED_DIM), jnp.float32),
        pltpu.SemaphoreType.DMA,
        pltpu.SemaphoreType.DMA,
    ],
)
def _sc_scatter(src_hbm, dst_hbm, y_hbm, zeros_hbm, out_hbm,
                src_v, dst_v, buf0, buf1, acc_sh, g0, g1):
    bufs = (buf0, buf1)
    gsems = (g0, g1)
    c = lax.axis_index("c")
    s = lax.axis_index("s")
    wid = c * NS + s
    pltpu.sync_copy(zeros_hbm, acc_sh.at[pl.ds(s * RPT, RPT)])
    plsc.subcore_barrier()

    def _start_g(j, k):
        pltpu.async_copy(y_hbm.at[src_v.at[j]], bufs[k], gsems[k])

    def _wait_g(k):
        pltpu.make_async_copy(y_hbm.at[src_v.at[0]], bufs[k],
                              gsems[k]).wait()

    for h in range(2):
        pltpu.sync_copy(src_hbm.at[wid, h], src_v)
        pltpu.sync_copy(dst_hbm.at[wid, h], dst_v)
        _start_g(0, 0)

        def _round(i, carry):
            j0 = i * 2
            _wait_g(0)
            _start_g(j0 + 1, 1)
            pltpu.sync_copy(bufs[0], acc_sh.at[dst_v.at[j0]], add=True)
            _wait_g(1)

            @pl.when(i < CHH // 2 - 1)
            def _pg():
                _start_g(j0 + 2, 0)

            pltpu.sync_copy(bufs[1], acc_sh.at[dst_v.at[j0 + 1]],
                            add=True)
            return carry

        lax.fori_loop(0, CHH // 2, _round, 0)

    plsc.subcore_barrier()
    pltpu.sync_copy(acc_sh.at[pl.ds(s * RPT, RPT)],
                    out_hbm.at[c, pl.ds(s * RPT, RPT)])


def _tc_combine_body(acc_ref, y_ref, degp_ref, b_ref, out_ref):
    acc = acc_ref[0] + acc_ref[1]
    dinv = _dinv_col(degp_ref[...])
    out_ref[...] = dinv * (acc + y_ref[...]) + b_ref[...][None, :]


_tc_combine = pl.pallas_call(
    _tc_combine_body,
    grid=(NP // _TCR,),
    in_specs=[
        pl.BlockSpec((NC, _TCR, EMBED_DIM), lambda i: (0, i, 0)),
        pl.BlockSpec((_TCR, EMBED_DIM), lambda i: (i, 0)),
        pl.BlockSpec((NW, _TCR), lambda i: (0, i)),
        pl.BlockSpec((EMBED_DIM,), lambda i: (0,)),
    ],
    out_specs=pl.BlockSpec((_TCR, EMBED_DIM), lambda i: (i, 0)),
    out_shape=jax.ShapeDtypeStruct((NP, EMBED_DIM), jnp.float32),
)


@functools.partial(
    pl.kernel,
    out_type=jax.ShapeDtypeStruct((XROWS, 128, EMBED_DIM), jnp.float32),
    mesh=_MESH,
    scratch_types=[
        pltpu.VMEM((XPW, 128), jnp.int32),
        pltpu.VMEM((128, EMBED_DIM), jnp.float32),
    ],
)
def _sc_gather(emb_hbm, x_hbm, out_hbm, idx_v, rows_v):
    c = lax.axis_index("c")
    s = lax.axis_index("s")
    wid = c * NS + s
    pltpu.sync_copy(x_hbm.at[wid], idx_v)

    def _chunk(j, carry):
        pltpu.sync_copy(emb_hbm.at[idx_v.at[j]], rows_v)
        pltpu.sync_copy(rows_v, out_hbm.at[wid * XPW + j])
        return carry

    lax.fori_loop(0, XPW, _chunk, 0)


def kernel(x, features, edge_index, W, b):
    src = edge_index[0].astype(jnp.int32)
    dst = edge_index[1].astype(jnp.int32)
    npad = EPAD - N_EDGES
    pad_dst = N_NODES + (lax.iota(jnp.int32, npad) % (NP - N_NODES))
    pad_src = lax.rem(N_NODES + lax.iota(jnp.int32, npad), NP)
    src2d = jnp.concatenate(
        [src, pad_src]).reshape(NW, 2, CHH, 128)
    dst2d = jnp.concatenate(
        [dst, pad_dst]).reshape(NW, 2, CHH, 128)
    x2d = x.astype(jnp.int32).reshape(NW, XPW, 128)
    zeros_blk = jnp.zeros((RPT, EMBED_DIM), jnp.float32)
    feat_pad = jnp.concatenate(
        [features, jnp.zeros((NP - N_NODES, D_FEAT), jnp.float32)])

    degp = _sc_degree(dst2d).reshape(NW, NP)
    y = _tc_prep(feat_pad, W, degp)
    acc = _sc_scatter(src2d, dst2d, y, zeros_blk)
    emb = _tc_combine(acc, y, degp, b)
    out = _sc_gather(emb, x2d)
    return out.reshape(BATCH, NUM_FIELDS, EMBED_DIM)


# speedup vs baseline: 1.9849x; 1.0238x over previous
import functools

import jax
import jax.numpy as jnp
from jax import lax
from jax.experimental import pallas as pl
from jax.experimental.pallas import tpu as pltpu
from jax.experimental.pallas import tpu_sc as plsc

N_NODES = 10000
N_EDGES = 320000
D_FEAT = 128
EMBED_DIM = 128
BATCH = 4096
NUM_FIELDS = 26

NC = 2
NS = 16
NW = NC * NS
L = 16

CH = 80
EPW = CH * 128
EPAD = NW * EPW
NBUF = 2
CHH = CH // 2
NP = 10240
RPT = NP // NS
DUMMY = N_NODES + 16
XROWS = BATCH * NUM_FIELDS // 128
XPW = XROWS // NW

_MESH = plsc.VectorSubcoreMesh(core_axis_name="c", subcore_axis_name="s")


@functools.partial(
    pl.kernel,
    out_type=jax.ShapeDtypeStruct((NW * NP,), jnp.float32),
    mesh=_MESH,
    scratch_types=[
        pltpu.VMEM((2, CHH, 128), jnp.int32),
        pltpu.VMEM((NP,), jnp.float32),
    ],
    compiler_params=pltpu.CompilerParams(needs_layout_passes=False),
)
def _sc_degree(dst_hbm, out_hbm, idx_v, deg_v):
    c = lax.axis_index("c")
    s = lax.axis_index("s")
    wid = c * NS + s
    pltpu.sync_copy(dst_hbm.at[wid], idx_v)

    zeros = jnp.zeros((L,), jnp.float32)

    def _zero(i, carry):
        deg_v[pl.ds(i * L, L)] = zeros
        return carry

    lax.fori_loop(0, NP // L, _zero, 0)

    ones = jnp.ones((L,), jnp.float32)

    for h in range(2):
        def _chunk(j, carry):
            def _sub(k, carry2):
                idx16 = idx_v[h, j, pl.ds(k * L, L)]
                plsc.addupdate_scatter(deg_v, [idx16], ones)
                return carry2

            return lax.fori_loop(0, 128 // L, _sub, carry)

        lax.fori_loop(0, CHH, _chunk, 0)
    pltpu.sync_copy(deg_v, out_hbm.at[pl.ds(wid * NP, NP)])


def _dinv_col(degp_blk):
    ones = jnp.ones((NW, 1), jnp.float32)
    deg = lax.dot_general(degp_blk, ones, (((0,), (0,)), ((), ())),
                          preferred_element_type=jnp.float32) + 1.0
    return lax.rsqrt(deg)


def _tc_prep_body(feat_ref, w_ref, degp_ref, y_ref):
    xw = jnp.dot(feat_ref[...], w_ref[...], preferred_element_type=jnp.float32)
    y_ref[...] = xw * _dinv_col(degp_ref[...])


_TCR = 1024

_tc_prep = pl.pallas_call(
    _tc_prep_body,
    grid=(NP // _TCR,),
    in_specs=[
        pl.BlockSpec((_TCR, D_FEAT), lambda i: (i, 0)),
        pl.BlockSpec((D_FEAT, EMBED_DIM), lambda i: (0, 0)),
        pl.BlockSpec((NW, _TCR), lambda i: (0, i)),
    ],
    out_specs=pl.BlockSpec((_TCR, EMBED_DIM), lambda i: (i, 0)),
    out_shape=jax.ShapeDtypeStruct((NP, EMBED_DIM), jnp.float32),
)


@functools.partial(
    pl.kernel,
    out_type=jax.ShapeDtypeStruct((NC, NP, EMBED_DIM), jnp.float32),
    mesh=_MESH,
    scratch_types=[
        pltpu.VMEM((CHH, 128), jnp.int32),
        pltpu.VMEM((CHH, 128), jnp.int32),
        pltpu.VMEM((128, EMBED_DIM), jnp.float32),
        pltpu.VMEM((128, EMBED_DIM), jnp.float32),
        pltpu.VMEM_SHARED((NP, EMBED_DIM), jnp.float32),
        pltpu.SemaphoreType.DMA,
        pltpu.SemaphoreType.DMA,
    ],
)
def _sc_scatter(src_hbm, dst_hbm, y_hbm, zeros_hbm, out_hbm,
                src_v, dst_v, buf0, buf1, acc_sh, g0, g1):
    bufs = (buf0, buf1)
    gsems = (g0, g1)
    c = lax.axis_index("c")
    s = lax.axis_index("s")
    wid = c * NS + s
    pltpu.sync_copy(zeros_hbm, acc_sh.at[pl.ds(s * RPT, RPT)])
    plsc.subcore_barrier()

    def _start_g(j, k):
        pltpu.async_copy(y_hbm.at[src_v.at[j]], bufs[k], gsems[k])

    def _wait_g(k):
        pltpu.make_async_copy(y_hbm.at[src_v.at[0]], bufs[k],
                              gsems[k]).wait()

    for h in range(2):
        pltpu.sync_copy(src_hbm.at[wid, h], src_v)
        pltpu.sync_copy(dst_hbm.at[wid, h], dst_v)
        _start_g(0, 0)

        def _round(i, carry):
            j0 = i * 2
            _wait_g(0)
            _start_g(j0 + 1, 1)
            pltpu.sync_copy(bufs[0], acc_sh.at[dst_v.at[j0]], add=True)
            _wait_g(1)

            @pl.when(i < CHH // 2 - 1)
            def _pg():
                _start_g(j0 + 2, 0)

            pltpu.sync_copy(bufs[1], acc_sh.at[dst_v.at[j0 + 1]],
                            add=True)
            return carry

        lax.fori_loop(0, CHH // 2, _round, 0)

    plsc.subcore_barrier()
    pltpu.sync_copy(acc_sh.at[pl.ds(s * RPT, RPT)],
                    out_hbm.at[c, pl.ds(s * RPT, RPT)])


def _tc_combine_body(acc_ref, y_ref, degp_ref, b_ref, out_ref):
    acc = acc_ref[0] + acc_ref[1]
    dinv = _dinv_col(degp_ref[...])
    out_ref[...] = dinv * (acc + y_ref[...]) + b_ref[...][None, :]


_tc_combine = pl.pallas_call(
    _tc_combine_body,
    grid=(NP // _TCR,),
    in_specs=[
        pl.BlockSpec((NC, _TCR, EMBED_DIM), lambda i: (0, i, 0)),
        pl.BlockSpec((_TCR, EMBED_DIM), lambda i: (i, 0)),
        pl.BlockSpec((NW, _TCR), lambda i: (0, i)),
        pl.BlockSpec((EMBED_DIM,), lambda i: (0,)),
    ],
    out_specs=pl.BlockSpec((_TCR, EMBED_DIM), lambda i: (i, 0)),
    out_shape=jax.ShapeDtypeStruct((NP, EMBED_DIM), jnp.float32),
)


@functools.partial(
    pl.kernel,
    out_type=jax.ShapeDtypeStruct((XROWS, 128, EMBED_DIM), jnp.float32),
    mesh=_MESH,
    scratch_types=[
        pltpu.VMEM((XPW, 128), jnp.int32),
        pltpu.VMEM((128, EMBED_DIM), jnp.float32),
        pltpu.VMEM((128, EMBED_DIM), jnp.float32),
        pltpu.SemaphoreType.DMA,
        pltpu.SemaphoreType.DMA,
    ],
)
def _sc_gather(emb_hbm, x_hbm, out_hbm, idx_v, rows_v0, rows_v1, sem0,
               sem1):
    bufs = (rows_v0, rows_v1)
    sems = (sem0, sem1)
    c = lax.axis_index("c")
    s = lax.axis_index("s")
    wid = c * NS + s
    pltpu.sync_copy(x_hbm.at[wid], idx_v)

    def _start_g(j, k):
        pltpu.async_copy(emb_hbm.at[idx_v.at[j]], bufs[k], sems[k])

    def _wait_g(k):
        pltpu.make_async_copy(emb_hbm.at[idx_v.at[0]], bufs[k],
                              sems[k]).wait()

    _start_g(0, 0)

    def _chunk(i, carry):
        j0 = i * 2
        _wait_g(0)
        _start_g(j0 + 1, 1)
        pltpu.sync_copy(bufs[0], out_hbm.at[wid * XPW + j0])
        _wait_g(1)

        @pl.when(i < XPW // 2 - 1)
        def _pg():
            _start_g(j0 + 2, 0)

        pltpu.sync_copy(bufs[1], out_hbm.at[wid * XPW + j0 + 1])
        return carry

    lax.fori_loop(0, XPW // 2, _chunk, 0)


def kernel(x, features, edge_index, W, b):
    src = edge_index[0].astype(jnp.int32)
    dst = edge_index[1].astype(jnp.int32)
    npad = EPAD - N_EDGES
    pad_dst = N_NODES + (lax.iota(jnp.int32, npad) % (NP - N_NODES))
    pad_src = lax.rem(N_NODES + lax.iota(jnp.int32, npad), NP)
    src2d = jnp.concatenate(
        [src, pad_src]).reshape(NW, 2, CHH, 128)
    dst2d = jnp.concatenate(
        [dst, pad_dst]).reshape(NW, 2, CHH, 128)
    x2d = x.astype(jnp.int32).reshape(NW, XPW, 128)
    zeros_blk = jnp.zeros((RPT, EMBED_DIM), jnp.float32)
    feat_pad = jnp.concatenate(
        [features, jnp.zeros((NP - N_NODES, D_FEAT), jnp.float32)])

    degp = _sc_degree(dst2d).reshape(NW, NP)
    y = _tc_prep(feat_pad, W, degp)
    acc = _sc_scatter(src2d, dst2d, y, zeros_blk)
    emb = _tc_combine(acc, y, degp, b)
    out = _sc_gather(emb, x2d)
    return out.reshape(BATCH, NUM_FIELDS, EMBED_DIM)


# speedup vs baseline: 1.9857x; 1.0004x over previous
import functools

import jax
import jax.numpy as jnp
from jax import lax
from jax.experimental import pallas as pl
from jax.experimental.pallas import tpu as pltpu
from jax.experimental.pallas import tpu_sc as plsc

N_NODES = 10000
N_EDGES = 320000
D_FEAT = 128
EMBED_DIM = 128
BATCH = 4096
NUM_FIELDS = 26

NC = 2
NS = 16
NW = NC * NS
L = 16

CH = 80
EPW = CH * 128
EPAD = NW * EPW
NBUF = 2
CHH = CH // 2
NP = 10240
RPT = NP // NS
DUMMY = N_NODES + 16
XROWS = BATCH * NUM_FIELDS // 128
XPW = XROWS // NW

_MESH = plsc.VectorSubcoreMesh(core_axis_name="c", subcore_axis_name="s")


@functools.partial(
    pl.kernel,
    out_type=jax.ShapeDtypeStruct((NW * NP,), jnp.float32),
    mesh=_MESH,
    scratch_types=[
        pltpu.VMEM((2, CHH, 128), jnp.int32),
        pltpu.VMEM((NP,), jnp.float32),
    ],
    compiler_params=pltpu.CompilerParams(needs_layout_passes=False),
)
def _sc_degree(dst_hbm, out_hbm, idx_v, deg_v):
    c = lax.axis_index("c")
    s = lax.axis_index("s")
    wid = c * NS + s
    pltpu.sync_copy(dst_hbm.at[wid], idx_v)

    zeros = jnp.zeros((L,), jnp.float32)

    def _zero(i, carry):
        deg_v[pl.ds(i * L, L)] = zeros
        return carry

    lax.fori_loop(0, NP // L, _zero, 0)

    ones = jnp.ones((L,), jnp.float32)

    for h in range(2):
        def _chunk(j, carry):
            def _sub(k, carry2):
                idx16 = idx_v[h, j, pl.ds(k * L, L)]
                plsc.addupdate_scatter(deg_v, [idx16], ones)
                return carry2

            return lax.fori_loop(0, 128 // L, _sub, carry)

        lax.fori_loop(0, CHH, _chunk, 0)
    pltpu.sync_copy(deg_v, out_hbm.at[pl.ds(wid * NP, NP)])


def _dinv_col(degp_blk):
    ones = jnp.ones((NW, 1), jnp.float32)
    deg = lax.dot_general(degp_blk, ones, (((0,), (0,)), ((), ())),
                          preferred_element_type=jnp.float32) + 1.0
    return lax.rsqrt(deg)


def _tc_prep_body(feat_ref, w_ref, degp_ref, y_ref):
    xw = jnp.dot(feat_ref[...], w_ref[...], preferred_element_type=jnp.float32)
    y_ref[...] = xw * _dinv_col(degp_ref[...])


_TCR = 1024

_tc_prep = pl.pallas_call(
    _tc_prep_body,
    grid=(NP // _TCR,),
    in_specs=[
        pl.BlockSpec((_TCR, D_FEAT), lambda i: (i, 0)),
        pl.BlockSpec((D_FEAT, EMBED_DIM), lambda i: (0, 0)),
        pl.BlockSpec((NW, _TCR), lambda i: (0, i)),
    ],
    out_specs=pl.BlockSpec((_TCR, EMBED_DIM), lambda i: (i, 0)),
    out_shape=jax.ShapeDtypeStruct((NP, EMBED_DIM), jnp.float32),
)


@functools.partial(
    pl.kernel,
    out_type=jax.ShapeDtypeStruct((NC, NP, EMBED_DIM), jnp.float32),
    mesh=_MESH,
    scratch_types=[
        pltpu.VMEM((CHH, 128), jnp.int32),
        pltpu.VMEM((CHH, 128), jnp.int32),
        pltpu.VMEM((128, EMBED_DIM), jnp.float32),
        pltpu.VMEM((128, EMBED_DIM), jnp.float32),
        pltpu.VMEM_SHARED((NP, EMBED_DIM), jnp.float32),
        pltpu.SemaphoreType.DMA,
        pltpu.SemaphoreType.DMA,
    ],
)
def _sc_scatter(src_hbm, dst_hbm, y_hbm, zeros_hbm, out_hbm,
                src_v, dst_v, buf0, buf1, acc_sh, g0, g1):
    bufs = (buf0, buf1)
    gsems = (g0, g1)
    c = lax.axis_index("c")
    s = lax.axis_index("s")
    wid = c * NS + s
    pltpu.sync_copy(zeros_hbm, acc_sh.at[pl.ds(s * RPT, RPT)])
    plsc.subcore_barrier()

    def _start_g(j, k):
        pltpu.async_copy(y_hbm.at[src_v.at[j]], bufs[k], gsems[k])

    def _wait_g(k):
        pltpu.make_async_copy(y_hbm.at[src_v.at[0]], bufs[k],
                              gsems[k]).wait()

    for h in range(2):
        pltpu.sync_copy(src_hbm.at[wid, h], src_v)
        pltpu.sync_copy(dst_hbm.at[wid, h], dst_v)
        _start_g(0, 0)

        def _round(i, carry):
            j0 = i * 2
            _wait_g(0)
            _start_g(j0 + 1, 1)
            pltpu.sync_copy(bufs[0], acc_sh.at[dst_v.at[j0]], add=True)
            _wait_g(1)

            @pl.when(i < CHH // 2 - 1)
            def _pg():
                _start_g(j0 + 2, 0)

            pltpu.sync_copy(bufs[1], acc_sh.at[dst_v.at[j0 + 1]],
                            add=True)
            return carry

        lax.fori_loop(0, CHH // 2, _round, 0)

    plsc.subcore_barrier()
    pltpu.sync_copy(acc_sh.at[pl.ds(s * RPT, RPT)],
                    out_hbm.at[c, pl.ds(s * RPT, RPT)])


def _tc_combine_body(acc_ref, y_ref, degp_ref, b_ref, out_ref):
    acc = acc_ref[0] + acc_ref[1]
    dinv = _dinv_col(degp_ref[...])
    out_ref[...] = dinv * (acc + y_ref[...]) + b_ref[...][None, :]


_tc_combine = pl.pallas_call(
    _tc_combine_body,
    grid=(NP // _TCR,),
    in_specs=[
        pl.BlockSpec((NC, _TCR, EMBED_DIM), lambda i: (0, i, 0)),
        pl.BlockSpec((_TCR, EMBED_DIM), lambda i: (i, 0)),
        pl.BlockSpec((NW, _TCR), lambda i: (0, i)),
        pl.BlockSpec((EMBED_DIM,), lambda i: (0,)),
    ],
    out_specs=pl.BlockSpec((_TCR, EMBED_DIM), lambda i: (i, 0)),
    out_shape=jax.ShapeDtypeStruct((NP, EMBED_DIM), jnp.float32),
)


@functools.partial(
    pl.kernel,
    out_type=jax.ShapeDtypeStruct((XROWS, 128, EMBED_DIM), jnp.float32),
    mesh=_MESH,
    scratch_types=[
        pltpu.VMEM((XPW, 128), jnp.int32),
        pltpu.VMEM((128, EMBED_DIM), jnp.float32),
        pltpu.VMEM((128, EMBED_DIM), jnp.float32),
        pltpu.SemaphoreType.DMA,
        pltpu.SemaphoreType.DMA,
    ],
)
def _sc_gather(emb_hbm, x_hbm, out_hbm, idx_v, rows_v0, rows_v1, sem0,
               sem1):
    bufs = (rows_v0, rows_v1)
    sems = (sem0, sem1)
    c = lax.axis_index("c")
    s = lax.axis_index("s")
    wid = c * NS + s
    pltpu.sync_copy(x_hbm.at[wid], idx_v)

    def _start_g(j, k):
        pltpu.async_copy(emb_hbm.at[idx_v.at[j]], bufs[k], sems[k])

    def _wait_g(k):
        pltpu.make_async_copy(emb_hbm.at[idx_v.at[0]], bufs[k],
                              sems[k]).wait()

    _start_g(0, 0)

    def _chunk(i, carry):
        j0 = i * 2
        _wait_g(0)
        _start_g(j0 + 1, 1)
        pltpu.sync_copy(bufs[0], out_hbm.at[wid * XPW + j0])
        _wait_g(1)

        @pl.when(i < XPW // 2 - 1)
        def _pg():
            _start_g(j0 + 2, 0)

        pltpu.sync_copy(bufs[1], out_hbm.at[wid * XPW + j0 + 1])
        return carry

    lax.fori_loop(0, XPW // 2, _chunk, 0)


def kernel(x, features, edge_index, W, b):
    src = edge_index[0].astype(jnp.int32)
    dst = edge_index[1].astype(jnp.int32)
    npad = EPAD - N_EDGES
    pad_dst = N_NODES + (lax.iota(jnp.int32, npad) % (NP - N_NODES))
    pad_src = lax.iota(jnp.int32, npad) % N_NODES
    src2d = jnp.concatenate(
        [src, pad_src]).reshape(NW, 2, CHH, 128)
    dst2d = jnp.concatenate(
        [dst, pad_dst]).reshape(NW, 2, CHH, 128)
    x2d = x.astype(jnp.int32).reshape(NW, XPW, 128)
    zeros_blk = jnp.zeros((RPT, EMBED_DIM), jnp.float32)

    degp = _sc_degree(dst2d).reshape(NW, NP)
    y = _tc_prep(features, W, degp)
    acc = _sc_scatter(src2d, dst2d, y, zeros_blk)
    emb = _tc_combine(acc, y, degp, b)
    out = _sc_gather(emb, x2d)
    return out.reshape(BATCH, NUM_FIELDS, EMBED_DIM)


# speedup vs baseline: 1.9961x; 1.0052x over previous
import functools

import jax
import jax.numpy as jnp
from jax import lax
from jax.experimental import pallas as pl
from jax.experimental.pallas import tpu as pltpu
from jax.experimental.pallas import tpu_sc as plsc

N_NODES = 10000
N_EDGES = 320000
D_FEAT = 128
EMBED_DIM = 128
BATCH = 4096
NUM_FIELDS = 26

NC = 2
NS = 16
NW = NC * NS
L = 16

CH = 80
EPW = CH * 128
EPAD = NW * EPW
CHH = CH // 2
NP = 10240
RPT = NP // NS
XROWS = BATCH * NUM_FIELDS // 128
XPW = XROWS // NW

_MESH = plsc.VectorSubcoreMesh(core_axis_name="c", subcore_axis_name="s")


@functools.partial(
    pl.kernel,
    out_type=jax.ShapeDtypeStruct((NW * NP,), jnp.float32),
    mesh=_MESH,
    scratch_types=[
        pltpu.VMEM((2, CHH, 128), jnp.int32),
        pltpu.VMEM((NP,), jnp.float32),
    ],
    compiler_params=pltpu.CompilerParams(needs_layout_passes=False),
)
def _sc_degree(dst_hbm, out_hbm, idx_v, deg_v):
    c = lax.axis_index("c")
    s = lax.axis_index("s")
    wid = c * NS + s
    pltpu.sync_copy(dst_hbm.at[wid], idx_v)

    zeros = jnp.zeros((L,), jnp.float32)

    def _zero(i, carry):
        deg_v[pl.ds(i * L, L)] = zeros
        return carry

    lax.fori_loop(0, NP // L, _zero, 0)

    ones = jnp.ones((L,), jnp.float32)

    for h in range(2):
        def _chunk(j, carry):
            def _sub(k, carry2):
                idx16 = idx_v[h, j, pl.ds(k * L, L)]
                plsc.addupdate_scatter(deg_v, [idx16], ones)
                return carry2

            return lax.fori_loop(0, 128 // L, _sub, carry)

        lax.fori_loop(0, CHH, _chunk, 0)
    pltpu.sync_copy(deg_v, out_hbm.at[pl.ds(wid * NP, NP)])


def _dinv_col(degp_blk):
    ones = jnp.ones((NW, 1), jnp.float32)
    deg = lax.dot_general(degp_blk, ones, (((0,), (0,)), ((), ())),
                          preferred_element_type=jnp.float32) + 1.0
    return lax.rsqrt(deg)


def _tc_prep_body(feat_ref, w_ref, degp_ref, y_ref):
    xw = jnp.dot(feat_ref[...], w_ref[...], preferred_element_type=jnp.float32)
    y_ref[...] = xw * _dinv_col(degp_ref[...])


_TCR = 1024

_tc_prep = pl.pallas_call(
    _tc_prep_body,
    grid=(NP // _TCR,),
    in_specs=[
        pl.BlockSpec((_TCR, D_FEAT), lambda i: (i, 0)),
        pl.BlockSpec((D_FEAT, EMBED_DIM), lambda i: (0, 0)),
        pl.BlockSpec((NW, _TCR), lambda i: (0, i)),
    ],
    out_specs=pl.BlockSpec((_TCR, EMBED_DIM), lambda i: (i, 0)),
    out_shape=jax.ShapeDtypeStruct((NP, EMBED_DIM), jnp.float32),
)


@functools.partial(
    pl.kernel,
    out_type=jax.ShapeDtypeStruct((NC, NP, EMBED_DIM), jnp.float32),
    mesh=_MESH,
    scratch_types=[
        pltpu.VMEM((CHH, 128), jnp.int32),
        pltpu.VMEM((CHH, 128), jnp.int32),
        pltpu.VMEM((128, EMBED_DIM), jnp.float32),
        pltpu.VMEM((128, EMBED_DIM), jnp.float32),
        pltpu.VMEM_SHARED((NP, EMBED_DIM), jnp.float32),
        pltpu.SemaphoreType.DMA,
        pltpu.SemaphoreType.DMA,
    ],
)
def _sc_scatter(src_hbm, dst_hbm, y_hbm, zeros_hbm, out_hbm,
                src_v, dst_v, buf0, buf1, acc_sh, g0, g1):
    bufs = (buf0, buf1)
    gsems = (g0, g1)
    c = lax.axis_index("c")
    s = lax.axis_index("s")
    wid = c * NS + s
    pltpu.sync_copy(zeros_hbm, acc_sh.at[pl.ds(s * RPT, RPT)])
    plsc.subcore_barrier()

    def _start_g(j, k):
        pltpu.async_copy(y_hbm.at[src_v.at[j]], bufs[k], gsems[k])

    def _wait_g(k):
        pltpu.make_async_copy(y_hbm.at[src_v.at[0]], bufs[k],
                              gsems[k]).wait()

    for h in range(2):
        pltpu.sync_copy(src_hbm.at[wid, h], src_v)
        pltpu.sync_copy(dst_hbm.at[wid, h], dst_v)
        _start_g(0, 0)

        def _round(i, carry):
            j0 = i * 2
            _wait_g(0)
            _start_g(j0 + 1, 1)
            pltpu.sync_copy(bufs[0], acc_sh.at[dst_v.at[j0]], add=True)
            _wait_g(1)

            @pl.when(i < CHH // 2 - 1)
            def _pg():
                _start_g(j0 + 2, 0)

            pltpu.sync_copy(bufs[1], acc_sh.at[dst_v.at[j0 + 1]],
                            add=True)
            return carry

        lax.fori_loop(0, CHH // 2, _round, 0)

    plsc.subcore_barrier()
    pltpu.sync_copy(acc_sh.at[pl.ds(s * RPT, RPT)],
                    out_hbm.at[c, pl.ds(s * RPT, RPT)])


def _tc_combine_body(acc_ref, y_ref, degp_ref, b_ref, out_ref):
    acc = acc_ref[0] + acc_ref[1]
    dinv = _dinv_col(degp_ref[...])
    out_ref[...] = dinv * (acc + y_ref[...]) + b_ref[...][None, :]


_tc_combine = pl.pallas_call(
    _tc_combine_body,
    grid=(NP // _TCR,),
    in_specs=[
        pl.BlockSpec((NC, _TCR, EMBED_DIM), lambda i: (0, i, 0)),
        pl.BlockSpec((_TCR, EMBED_DIM), lambda i: (i, 0)),
        pl.BlockSpec((NW, _TCR), lambda i: (0, i)),
        pl.BlockSpec((EMBED_DIM,), lambda i: (0,)),
    ],
    out_specs=pl.BlockSpec((_TCR, EMBED_DIM), lambda i: (i, 0)),
    out_shape=jax.ShapeDtypeStruct((NP, EMBED_DIM), jnp.float32),
)


@functools.partial(
    pl.kernel,
    out_type=jax.ShapeDtypeStruct((XROWS, 128, EMBED_DIM), jnp.float32),
    mesh=_MESH,
    scratch_types=[
        pltpu.VMEM((XPW, 128), jnp.int32),
        pltpu.VMEM((128, EMBED_DIM), jnp.float32),
        pltpu.VMEM((128, EMBED_DIM), jnp.float32),
        pltpu.SemaphoreType.DMA,
        pltpu.SemaphoreType.DMA,
    ],
)
def _sc_gather(emb_hbm, x_hbm, out_hbm, idx_v, rows_v0, rows_v1, sem0,
               sem1):
    bufs = (rows_v0, rows_v1)
    sems = (sem0, sem1)
    c = lax.axis_index("c")
    s = lax.axis_index("s")
    wid = c * NS + s
    pltpu.sync_copy(x_hbm.at[wid], idx_v)

    def _start_g(j, k):
        pltpu.async_copy(emb_hbm.at[idx_v.at[j]], bufs[k], sems[k])

    def _wait_g(k):
        pltpu.make_async_copy(emb_hbm.at[idx_v.at[0]], bufs[k],
                              sems[k]).wait()

    _start_g(0, 0)

    def _chunk(i, carry):
        j0 = i * 2
        _wait_g(0)
        _start_g(j0 + 1, 1)
        pltpu.sync_copy(bufs[0], out_hbm.at[wid * XPW + j0])
        _wait_g(1)

        @pl.when(i < XPW // 2 - 1)
        def _pg():
            _start_g(j0 + 2, 0)

        pltpu.sync_copy(bufs[1], out_hbm.at[wid * XPW + j0 + 1])
        return carry

    lax.fori_loop(0, XPW // 2, _chunk, 0)


def kernel(x, features, edge_index, W, b):
    src = edge_index[0].astype(jnp.int32)
    dst = edge_index[1].astype(jnp.int32)
    npad = EPAD - N_EDGES
    pad_dst = N_NODES + (lax.iota(jnp.int32, npad) % (NP - N_NODES))
    pad_src = lax.iota(jnp.int32, npad) % N_NODES
    src2d = jnp.concatenate(
        [src, pad_src]).reshape(NW, 2, CHH, 128)
    dst2d = jnp.concatenate(
        [dst, pad_dst]).reshape(NW, 2, CHH, 128)
    x2d = x.astype(jnp.int32).reshape(NW, XPW, 128)
    zeros_blk = jnp.zeros((RPT, EMBED_DIM), jnp.float32)

    degp = _sc_degree(dst2d).reshape(NW, NP)
    y = _tc_prep(features, W, degp)
    acc = _sc_scatter(src2d, dst2d, y, zeros_blk)
    emb = _tc_combine(acc, y, degp, b)
    out = _sc_gather(emb, x2d)
    return out.reshape(BATCH, NUM_FIELDS, EMBED_DIM)
